# Initial kernel scaffold; baseline (speedup 1.0000x reference)
import jax
import jax.numpy as jnp
from jax.experimental import pallas as pl


def kernel(vectors_init, W1_all, b1_all, W2_all, b2_all, eval_W, eval_b, pos, neg, target, rule_steps, ind_steps, pars_ind_steps, mask_idx):
    raise NotImplementedError("write your pallas kernel here")


# speedup vs baseline: 1.1677x; 1.1677x over previous
import functools

import jax
import jax.numpy as jnp
from jax import lax
from jax.experimental import pallas as pl
from jax.experimental.pallas import tpu as pltpu
from jax.experimental.pallas import tpu_sc as plsc

N = 100000
EMB = 128
S = 16
STEP = 4096
M = 50000
POS_WEIGHT = 2.0

NP = N + 8
DUMP = N
NC = 2
NS = 16
NW = NC * NS
CHUNK = 128

FCH = 13
M_PAD = NW * FCH * CHUNK

_mesh = plsc.VectorSubcoreMesh(core_axis_name="c", subcore_axis_name="s")


def _wid():
    return lax.axis_index("s") * NC + lax.axis_index("c")


@functools.partial(
    pl.kernel, mesh=_mesh,
    out_type=jax.ShapeDtypeStruct((2 * STEP, EMB), jnp.float32),
    scratch_types=[
        pltpu.VMEM((2, CHUNK), jnp.int32),
        pltpu.VMEM((2 * CHUNK, EMB), jnp.float32),
        pltpu.SemaphoreType.DMA,
        pltpu.SemaphoreType.DMA,
    ],
)
def _sc_gather_step(vec_hbm, idx_hbm, out_hbm, idx_v, rows_v, s0, s1):
    w = _wid()
    pltpu.sync_copy(idx_hbm.at[w], idx_v)
    c0 = pltpu.async_copy(vec_hbm.at[idx_v.at[0]], rows_v.at[pl.ds(0, CHUNK)], s0)
    c1 = pltpu.async_copy(vec_hbm.at[idx_v.at[1]], rows_v.at[pl.ds(CHUNK, CHUNK)], s1)
    c0.wait()
    c1.wait()
    pltpu.sync_copy(rows_v, out_hbm.at[pl.ds(w * 2 * CHUNK, 2 * CHUNK)])


@functools.partial(
    pl.kernel, mesh=_mesh,
    out_type=(),
    scratch_types=[
        pltpu.VMEM((1, CHUNK), jnp.int32),
        pltpu.VMEM((CHUNK, EMB), jnp.float32),
        pltpu.SemaphoreType.DMA,
    ],
)
def _sc_scatter_step(vec_hbm, rows_hbm, idx_hbm, idx_v, rows_v, s0):
    w = _wid()
    pltpu.sync_copy(idx_hbm.at[w], idx_v)
    pltpu.sync_copy(rows_hbm.at[pl.ds(w * CHUNK, CHUNK)], rows_v)
    pltpu.async_copy(rows_v, vec_hbm.at[idx_v.at[0]], s0).wait()


@functools.partial(
    pl.kernel, mesh=_mesh,
    out_type=(
        jax.ShapeDtypeStruct((M_PAD, EMB), jnp.float32),
        jax.ShapeDtypeStruct((M_PAD, EMB), jnp.float32),
    ),
    scratch_types=[
        pltpu.VMEM((FCH, CHUNK), jnp.int32),
        pltpu.VMEM((CHUNK, EMB), jnp.float32),
        pltpu.VMEM((CHUNK, EMB), jnp.float32),
        pltpu.SemaphoreType.DMA,
        pltpu.SemaphoreType.DMA,
    ],
)
def _sc_gather_final(vec_hbm, pnt_hbm, idx_hbm, vm_hbm, pg_hbm,
                     idx_v, rows_v, p_v, s0, s1):
    w = _wid()
    pltpu.sync_copy(idx_hbm.at[w], idx_v)
    for j in range(FCH):
        base = (w * FCH + j) * CHUNK
        cv = pltpu.async_copy(vec_hbm.at[idx_v.at[j]], rows_v, s0)
        cp = pltpu.async_copy(pnt_hbm.at[idx_v.at[j]], p_v, s1)
        cv.wait()
        cp.wait()
        pltpu.sync_copy(rows_v, vm_hbm.at[pl.ds(base, CHUNK)])
        pltpu.sync_copy(p_v, pg_hbm.at[pl.ds(base, CHUNK)])


_MLP_BLK = 1024


def _mlp_body(x_ref, w1_ref, b1_ref, w2_ref, b2_ref, o_ref):
    h = jnp.tanh(
        jnp.dot(x_ref[...], w1_ref[...], preferred_element_type=jnp.float32)
        + b1_ref[...]
    )
    o_ref[...] = (
        jnp.dot(h, w2_ref[...], preferred_element_type=jnp.float32)
        + b2_ref[...]
    )


_mlp = pl.pallas_call(
    _mlp_body,
    grid=(STEP // _MLP_BLK,),
    in_specs=[
        pl.BlockSpec((_MLP_BLK, 2 * EMB), lambda i: (i, 0)),
        pl.BlockSpec((2 * EMB, EMB), lambda i: (0, 0)),
        pl.BlockSpec((1, EMB), lambda i: (0, 0)),
        pl.BlockSpec((EMB, EMB), lambda i: (0, 0)),
        pl.BlockSpec((1, EMB), lambda i: (0, 0)),
    ],
    out_specs=pl.BlockSpec((_MLP_BLK, EMB), lambda i: (i, 0)),
    out_shape=jax.ShapeDtypeStruct((STEP, EMB), jnp.float32),
)


_LOSS_BLK = 2048


def _loss_body(vm_ref, pg_ref, ew_ref, eb_ref, o_ref):
    i = pl.program_id(0)
    vals = (
        jnp.dot(vm_ref[...], ew_ref[...], preferred_element_type=jnp.float32)
        + eb_ref[0]
    )
    rowi = i * _LOSS_BLK + lax.broadcasted_iota(jnp.int32, (_LOSS_BLK, 1), 0)
    valid = rowi < M
    p = jnp.where(valid, pg_ref[:, 0:1], 0.0)
    n = jnp.where(valid, pg_ref[:, 1:2], 0.0)
    t = pg_ref[:, 2:3]
    sg = jnp.clip(jax.nn.sigmoid(vals), 1e-07, 1.0 - 1e-07)
    contrib = -POS_WEIGHT * t * jnp.log(sg) - (1.0 - t) * jnp.log(1.0 - sg)
    part_loss = jnp.sum((p + n) * contrib)
    part_pos = jnp.sum(p * (vals >= 0.0).astype(jnp.float32))
    part_neg = jnp.sum(n * (vals < 0.0).astype(jnp.float32))

    @pl.when(i == 0)
    def _():
        o_ref[0] = 0.0
        o_ref[1] = 0.0
        o_ref[2] = 0.0

    o_ref[0] += part_loss
    o_ref[1] += part_pos
    o_ref[2] += part_neg


_loss = pl.pallas_call(
    _loss_body,
    grid=(M_PAD // _LOSS_BLK,),
    in_specs=[
        pl.BlockSpec((_LOSS_BLK, EMB), lambda i: (i, 0)),
        pl.BlockSpec((_LOSS_BLK, EMB), lambda i: (i, 0)),
        pl.BlockSpec((EMB, 1), lambda i: (0, 0)),
        pl.BlockSpec(memory_space=pltpu.SMEM),
    ],
    out_specs=pl.BlockSpec(memory_space=pltpu.SMEM),
    out_shape=jax.ShapeDtypeStruct((3,), jnp.float32),
)


def kernel(vectors_init, W1_all, b1_all, W2_all, b2_all, eval_W, eval_b,
           pos, neg, target, rule_steps, ind_steps, pars_ind_steps, mask_idx):
    i32 = jnp.int32
    f32 = jnp.float32

    vecpad = jnp.concatenate(
        [vectors_init, jnp.zeros((NP - N, EMB), f32)], axis=0)

    pnt = jnp.zeros((NP, EMB), f32)
    pnt = pnt.at[:N, 0].set(pos).at[:N, 1].set(neg).at[:N, 2].set(target)

    ind = ind_steps.astype(i32)
    posn = jnp.arange(STEP, dtype=i32)
    flat_idx = (jnp.arange(S, dtype=i32)[:, None] * NP + ind).reshape(-1)
    aux = jnp.full((S * NP,), -1, i32).at[flat_idx].max(
        jnp.tile(posn, S))
    winner = aux[flat_idx].reshape(S, STEP) == posn[None, :]
    ind_eff = jnp.where(winner, ind, DUMP).reshape(S, NW, 1, CHUNK)

    pars = pars_ind_steps.reshape(S, 2 * STEP).astype(i32)
    pars = pars.reshape(S, NW, 2, CHUNK)

    maskp = jnp.concatenate(
        [mask_idx.astype(i32), jnp.full((M_PAD - M,), DUMP, i32)]
    ).reshape(NW, FCH, CHUNK)

    vref = jax.new_ref(vecpad)

    parents = _sc_gather_step(vref, pars[0])
    for t in range(S):
        r = rule_steps[t]
        nr = _mlp(parents.reshape(STEP, 2 * EMB),
                  W1_all[r], b1_all[r][None, :], W2_all[r], b2_all[r][None, :])
        _sc_scatter_step(vref, nr, ind_eff[t])
        if t + 1 < S:
            parents = _sc_gather_step(vref, pars[t + 1])

    vm, pg = _sc_gather_final(vref, pnt, maskp)
    out3 = _loss(vm, pg, eval_W, eval_b)
    return (out3[0], out3[1], out3[2])


# speedup vs baseline: 1.6187x; 1.3863x over previous
import functools

import jax
import jax.numpy as jnp
from jax import lax
from jax.experimental import pallas as pl
from jax.experimental.pallas import tpu as pltpu
from jax.experimental.pallas import tpu_sc as plsc

N = 100000
EMB = 128
R = 8
S = 16
STEP = 4096
M = 50000
POS_WEIGHT = 2.0

NP = N + 8
DUMP = N
NC = 2
NS = 16
NW = NC * NS
CHUNK = 128

FCH = 13
M_PAD = NW * FCH * CHUNK

_mesh = plsc.VectorSubcoreMesh(core_axis_name="c", subcore_axis_name="s")


def _wid():
    return lax.axis_index("s") * NC + lax.axis_index("c")


@functools.partial(
    pl.kernel, mesh=_mesh,
    out_type=jax.ShapeDtypeStruct((2 * STEP, EMB), jnp.float32),
    scratch_types=[
        pltpu.VMEM((2, CHUNK), jnp.int32),
        pltpu.VMEM((2 * CHUNK, EMB), jnp.float32),
        pltpu.SemaphoreType.DMA,
        pltpu.SemaphoreType.DMA,
    ],
)
def _sc_gather_step(vec_hbm, idx_hbm, out_hbm, idx_v, rows_v, s0, s1):
    w = _wid()
    pltpu.sync_copy(idx_hbm.at[w], idx_v)
    c0 = pltpu.async_copy(vec_hbm.at[idx_v.at[0]], rows_v.at[pl.ds(0, CHUNK)], s0)
    c1 = pltpu.async_copy(vec_hbm.at[idx_v.at[1]], rows_v.at[pl.ds(CHUNK, CHUNK)], s1)
    c0.wait()
    c1.wait()
    pltpu.sync_copy(rows_v, out_hbm.at[pl.ds(w * 2 * CHUNK, 2 * CHUNK)])


@functools.partial(
    pl.kernel, mesh=_mesh,
    out_type=(),
    scratch_types=[
        pltpu.VMEM((1, CHUNK), jnp.int32),
        pltpu.VMEM((CHUNK, EMB), jnp.float32),
        pltpu.SemaphoreType.DMA,
    ],
)
def _sc_scatter_step(vec_hbm, rows_hbm, idx_hbm, idx_v, rows_v, s0):
    w = _wid()
    pltpu.sync_copy(idx_hbm.at[w], idx_v)
    pltpu.sync_copy(rows_hbm.at[pl.ds(w * CHUNK, CHUNK)], rows_v)
    pltpu.async_copy(rows_v, vec_hbm.at[idx_v.at[0]], s0).wait()


@functools.partial(
    pl.kernel, mesh=_mesh,
    out_type=(
        jax.ShapeDtypeStruct((M_PAD, EMB), jnp.float32),
        jax.ShapeDtypeStruct((M_PAD, EMB), jnp.float32),
    ),
    scratch_types=[
        pltpu.VMEM((FCH, CHUNK), jnp.int32),
        pltpu.VMEM((3, CHUNK, EMB), jnp.float32),
        pltpu.VMEM((3, CHUNK, EMB), jnp.float32),
        [pltpu.SemaphoreType.DMA] * 3,
        [pltpu.SemaphoreType.DMA] * 3,
        [pltpu.SemaphoreType.DMA] * 3,
        [pltpu.SemaphoreType.DMA] * 3,
    ],
)
def _sc_gather_final(vec_hbm, pnt_hbm, idx_hbm, vm_hbm, pg_hbm,
                     idx_v, rows_v, p_v, sv, sp, ov, op):
    w = _wid()
    pltpu.sync_copy(idx_hbm.at[w], idx_v)
    pend = {}
    for j in range(FCH):
        b = j % 3
        if j >= 3:
            pend[j - 3][2].wait()
            pend[j - 3][3].wait()
        gv = pltpu.async_copy(vec_hbm.at[idx_v.at[j]], rows_v.at[b], sv[b])
        gp = pltpu.async_copy(pnt_hbm.at[idx_v.at[j]], p_v.at[b], sp[b])
        pend[j] = [gv, gp, None, None]
        if j >= 1:
            k = j - 1
            bk = k % 3
            pend[k][0].wait()
            pend[k][1].wait()
            base = (w * FCH + k) * CHUNK
            pend[k][2] = pltpu.async_copy(
                rows_v.at[bk], vm_hbm.at[pl.ds(base, CHUNK)], ov[bk])
            pend[k][3] = pltpu.async_copy(
                p_v.at[bk], pg_hbm.at[pl.ds(base, CHUNK)], op[bk])
    k = FCH - 1
    bk = k % 3
    pend[k][0].wait()
    pend[k][1].wait()
    base = (w * FCH + k) * CHUNK
    pltpu.async_copy(rows_v.at[bk], vm_hbm.at[pl.ds(base, CHUNK)], ov[bk]).wait()
    pltpu.async_copy(p_v.at[bk], pg_hbm.at[pl.ds(base, CHUNK)], op[bk]).wait()
    for k in (FCH - 3, FCH - 2):
        pend[k][2].wait()
        pend[k][3].wait()


_MLP_BLK = 1024


def _mlp_body(r_ref, t_ref, b_ref, w1t_ref, w1b_ref, b1_ref, w2_ref, b2_ref,
              o_ref):
    del r_ref
    h = jnp.tanh(
        jnp.dot(t_ref[...], w1t_ref[0], preferred_element_type=jnp.float32)
        + jnp.dot(b_ref[...], w1b_ref[0], preferred_element_type=jnp.float32)
        + b1_ref[0, 0]
    )
    o_ref[...] = (
        jnp.dot(h, w2_ref[0], preferred_element_type=jnp.float32)
        + b2_ref[0, 0]
    )


_mlp = pl.pallas_call(
    _mlp_body,
    grid_spec=pltpu.PrefetchScalarGridSpec(
        num_scalar_prefetch=1,
        grid=(STEP // _MLP_BLK,),
        in_specs=[
            pl.BlockSpec((_MLP_BLK, EMB), lambda i, r: (i, 0)),
            pl.BlockSpec((_MLP_BLK, EMB), lambda i, r: (i + STEP // _MLP_BLK, 0)),
            pl.BlockSpec((1, EMB, EMB), lambda i, r: (r[0], 0, 0)),
            pl.BlockSpec((1, EMB, EMB), lambda i, r: (r[0], 1, 0)),
            pl.BlockSpec((1, 1, EMB), lambda i, r: (r[0], 0, 0)),
            pl.BlockSpec((1, EMB, EMB), lambda i, r: (r[0], 0, 0)),
            pl.BlockSpec((1, 1, EMB), lambda i, r: (r[0], 0, 0)),
        ],
        out_specs=pl.BlockSpec((_MLP_BLK, EMB), lambda i, r: (i, 0)),
    ),
    out_shape=jax.ShapeDtypeStruct((STEP, EMB), jnp.float32),
)


_LOSS_BLK = 2048


def _loss_body(vm_ref, pg_ref, ew_ref, eb_ref, o_ref):
    i = pl.program_id(0)
    vals = (
        jnp.dot(vm_ref[...], ew_ref[...], preferred_element_type=jnp.float32)
        + eb_ref[0]
    )
    rowi = i * _LOSS_BLK + lax.broadcasted_iota(jnp.int32, (_LOSS_BLK, 1), 0)
    valid = rowi < M
    p = jnp.where(valid, pg_ref[:, 0:1], 0.0)
    n = jnp.where(valid, pg_ref[:, 1:2], 0.0)
    t = pg_ref[:, 2:3]
    sg = jnp.clip(jax.nn.sigmoid(vals), 1e-07, 1.0 - 1e-07)
    contrib = -POS_WEIGHT * t * jnp.log(sg) - (1.0 - t) * jnp.log(1.0 - sg)
    part_loss = jnp.sum((p + n) * contrib)
    part_pos = jnp.sum(p * (vals >= 0.0).astype(jnp.float32))
    part_neg = jnp.sum(n * (vals < 0.0).astype(jnp.float32))

    @pl.when(i == 0)
    def _():
        o_ref[0] = 0.0
        o_ref[1] = 0.0
        o_ref[2] = 0.0

    o_ref[0] += part_loss
    o_ref[1] += part_pos
    o_ref[2] += part_neg


_loss = pl.pallas_call(
    _loss_body,
    grid=(M_PAD // _LOSS_BLK,),
    in_specs=[
        pl.BlockSpec((_LOSS_BLK, EMB), lambda i: (i, 0)),
        pl.BlockSpec((_LOSS_BLK, EMB), lambda i: (i, 0)),
        pl.BlockSpec((EMB, 1), lambda i: (0, 0)),
        pl.BlockSpec(memory_space=pltpu.SMEM),
    ],
    out_specs=pl.BlockSpec(memory_space=pltpu.SMEM),
    out_shape=jax.ShapeDtypeStruct((3,), jnp.float32),
)


def kernel(vectors_init, W1_all, b1_all, W2_all, b2_all, eval_W, eval_b,
           pos, neg, target, rule_steps, ind_steps, pars_ind_steps, mask_idx):
    i32 = jnp.int32
    f32 = jnp.float32

    vecpad = jnp.concatenate(
        [vectors_init, jnp.zeros((NP - N, EMB), f32)], axis=0)

    pnt = jnp.concatenate(
        [pos[:, None], neg[:, None], target[:, None],
         jnp.zeros((N, EMB - 3), f32)], axis=1)
    pnt = jnp.concatenate([pnt, jnp.zeros((NP - N, EMB), f32)], axis=0)

    ind = ind_steps.astype(i32)
    posn = jnp.arange(STEP, dtype=i32)
    flat_idx = (jnp.arange(S, dtype=i32)[:, None] * NP + ind).reshape(-1)
    aux = jnp.full((S * NP,), -1, i32).at[flat_idx].max(jnp.tile(posn, S))
    winner = aux[flat_idx].reshape(S, STEP) == posn[None, :]
    ind_eff = jnp.where(winner, ind, DUMP).reshape(S, NW, 1, CHUNK)

    pars = pars_ind_steps.astype(i32).transpose(0, 2, 1).reshape(S, 2 * STEP)
    pars = pars.reshape(S, NW, 2, CHUNK)

    maskp = jnp.concatenate(
        [mask_idx.astype(i32), jnp.full((M_PAD - M,), DUMP, i32)]
    ).reshape(NW, FCH, CHUNK)

    b1r = b1_all[:, None, :]
    b2r = b2_all[:, None, :]
    r_steps = rule_steps.astype(i32)
    vref = jax.new_ref(vecpad)

    parents = _sc_gather_step(vref, pars[0])
    for t in range(S):
        nr = _mlp(r_steps[t][None], parents, parents,
                  W1_all, W1_all, b1r, W2_all, b2r)
        _sc_scatter_step(vref, nr, ind_eff[t])
        if t + 1 < S:
            parents = _sc_gather_step(vref, pars[t + 1])

    vm, pg = _sc_gather_final(vref, pnt, maskp)
    out3 = _loss(vm, pg, eval_W, eval_b)
    return (out3[0], out3[1], out3[2])


# speedup vs baseline: 1.8717x; 1.1563x over previous
import functools

import jax
import jax.numpy as jnp
from jax import lax
from jax.experimental import pallas as pl
from jax.experimental.pallas import tpu as pltpu
from jax.experimental.pallas import tpu_sc as plsc

N = 100000
EMB = 128
R = 8
S = 16
STEP = 4096
M = 50000
POS_WEIGHT = 2.0

NP = 100352
NP2 = NP
DUMP = N
NC = 2
NS = 16
NW = NC * NS
CHUNK = 128

RNG = NP // NW
M_PAD = 50048

_mesh = plsc.VectorSubcoreMesh(core_axis_name="c", subcore_axis_name="s")


def _wid():
    return lax.axis_index("s") * NC + lax.axis_index("c")


@functools.partial(
    pl.kernel, mesh=_mesh,
    out_type=jax.ShapeDtypeStruct((2 * STEP, EMB), jnp.float32),
    scratch_types=[
        pltpu.VMEM((2, CHUNK), jnp.int32),
        pltpu.VMEM((2 * CHUNK, EMB), jnp.float32),
        pltpu.SemaphoreType.DMA,
        pltpu.SemaphoreType.DMA,
    ],
)
def _sc_gather_step(vec_hbm, idx_hbm, out_hbm, idx_v, rows_v, s0, s1):
    w = _wid()
    pltpu.sync_copy(idx_hbm.at[w], idx_v)
    c0 = pltpu.async_copy(vec_hbm.at[idx_v.at[0]], rows_v.at[pl.ds(0, CHUNK)], s0)
    c1 = pltpu.async_copy(vec_hbm.at[idx_v.at[1]], rows_v.at[pl.ds(CHUNK, CHUNK)], s1)
    c0.wait()
    c1.wait()
    pltpu.sync_copy(rows_v, out_hbm.at[pl.ds(w * 2 * CHUNK, 2 * CHUNK)])


@functools.partial(
    pl.kernel, mesh=_mesh,
    out_type=(),
    scratch_types=[
        pltpu.VMEM((1, CHUNK), jnp.int32),
        pltpu.VMEM((CHUNK, EMB), jnp.float32),
        pltpu.SemaphoreType.DMA,
    ],
)
def _sc_scatter_step(vec_hbm, rows_hbm, idx_hbm, idx_v, rows_v, s0):
    w = _wid()
    pltpu.sync_copy(idx_hbm.at[w], idx_v)
    pltpu.sync_copy(rows_hbm.at[pl.ds(w * CHUNK, CHUNK)], rows_v)
    pltpu.async_copy(rows_v, vec_hbm.at[idx_v.at[0]], s0).wait()


@functools.partial(
    pl.kernel, mesh=_mesh,
    out_type=jax.ShapeDtypeStruct((NP2,), jnp.float32),
    scratch_types=[
        pltpu.VMEM((M_PAD,), jnp.int32),
        pltpu.VMEM((RNG,), jnp.float32),
    ],
    compiler_params=pltpu.CompilerParams(needs_layout_passes=False),
)
def _sc_count(mask_hbm, cnt_hbm, mask_v, cnt_v):
    w = _wid()
    lo = w * RNG
    pltpu.sync_copy(mask_hbm, mask_v)

    def zbody(k, _):
        cnt_v[pl.ds(k * 16, 16)] = jnp.zeros((16,), jnp.float32)
        return 0

    lax.fori_loop(0, RNG // 16, zbody, 0)

    def body(k, _):
        base = k * 64
        for u in range(4):
            idx = mask_v[pl.ds(base + u * 16, 16)]
            inr = (idx >= lo) & (idx < lo + RNG)
            lidx = jnp.where(inr, idx - lo, 0)
            plsc.addupdate_scatter(
                cnt_v, [lidx], jnp.where(inr, 1.0, 0.0))
        return 0

    lax.fori_loop(0, M_PAD // 64, body, 0)
    pltpu.sync_copy(cnt_v, cnt_hbm.at[pl.ds(lo, RNG)])


_MLP_BLK = 1024


def _mlp_body(r_ref, t_ref, b_ref, w1t_ref, w1b_ref, b1_ref, w2_ref, b2_ref,
              o_ref):
    del r_ref
    h = jnp.tanh(
        jnp.dot(t_ref[...], w1t_ref[0], preferred_element_type=jnp.float32)
        + jnp.dot(b_ref[...], w1b_ref[0], preferred_element_type=jnp.float32)
        + b1_ref[0, 0]
    )
    o_ref[...] = (
        jnp.dot(h, w2_ref[0], preferred_element_type=jnp.float32)
        + b2_ref[0, 0]
    )


_mlp = pl.pallas_call(
    _mlp_body,
    grid_spec=pltpu.PrefetchScalarGridSpec(
        num_scalar_prefetch=1,
        grid=(STEP // _MLP_BLK,),
        in_specs=[
            pl.BlockSpec((_MLP_BLK, EMB), lambda i, r: (i, 0)),
            pl.BlockSpec((_MLP_BLK, EMB), lambda i, r: (i + STEP // _MLP_BLK, 0)),
            pl.BlockSpec((1, EMB, EMB), lambda i, r: (r[0], 0, 0)),
            pl.BlockSpec((1, EMB, EMB), lambda i, r: (r[0], 1, 0)),
            pl.BlockSpec((1, 1, EMB), lambda i, r: (r[0], 0, 0)),
            pl.BlockSpec((1, EMB, EMB), lambda i, r: (r[0], 0, 0)),
            pl.BlockSpec((1, 1, EMB), lambda i, r: (r[0], 0, 0)),
        ],
        out_specs=pl.BlockSpec((_MLP_BLK, EMB), lambda i, r: (i, 0)),
    ),
    out_shape=jax.ShapeDtypeStruct((STEP, EMB), jnp.float32),
)


_LOSS_BLK = 2048


def _loss_body(vec_ref, p_ref, n_ref, t_ref, c_ref, ew_ref, eb_ref, o_ref):
    i = pl.program_id(0)
    vals = (
        jnp.dot(vec_ref[...], ew_ref[...], preferred_element_type=jnp.float32)
        + eb_ref[0]
    )
    cp = c_ref[...] * p_ref[...]
    cn = c_ref[...] * n_ref[...]
    t = t_ref[...]
    sg = jnp.clip(jax.nn.sigmoid(vals), 1e-07, 1.0 - 1e-07)
    contrib = -POS_WEIGHT * t * jnp.log(sg) - (1.0 - t) * jnp.log(1.0 - sg)
    part_loss = jnp.sum((cp + cn) * contrib)
    part_pos = jnp.sum(cp * (vals >= 0.0).astype(jnp.float32))
    part_neg = jnp.sum(cn * (vals < 0.0).astype(jnp.float32))

    @pl.when(i == 0)
    def _():
        o_ref[0] = 0.0
        o_ref[1] = 0.0
        o_ref[2] = 0.0

    o_ref[0] += part_loss
    o_ref[1] += part_pos
    o_ref[2] += part_neg


_loss = pl.pallas_call(
    _loss_body,
    grid=(NP2 // _LOSS_BLK,),
    in_specs=[
        pl.BlockSpec((_LOSS_BLK, EMB), lambda i: (i, 0)),
        pl.BlockSpec((_LOSS_BLK, 1), lambda i: (i, 0)),
        pl.BlockSpec((_LOSS_BLK, 1), lambda i: (i, 0)),
        pl.BlockSpec((_LOSS_BLK, 1), lambda i: (i, 0)),
        pl.BlockSpec((_LOSS_BLK, 1), lambda i: (i, 0)),
        pl.BlockSpec((EMB, 1), lambda i: (0, 0)),
        pl.BlockSpec(memory_space=pltpu.SMEM),
    ],
    out_specs=pl.BlockSpec(memory_space=pltpu.SMEM),
    out_shape=jax.ShapeDtypeStruct((3,), jnp.float32),
)


def kernel(vectors_init, W1_all, b1_all, W2_all, b2_all, eval_W, eval_b,
           pos, neg, target, rule_steps, ind_steps, pars_ind_steps, mask_idx):
    i32 = jnp.int32
    f32 = jnp.float32

    vecpad = jnp.concatenate(
        [vectors_init, jnp.zeros((NP - N, EMB), f32)], axis=0)

    ind = ind_steps.astype(i32)
    posn = jnp.arange(STEP, dtype=i32)
    flat_idx = (jnp.arange(S, dtype=i32)[:, None] * NP + ind).reshape(-1)
    aux = jnp.full((S * NP,), -1, i32).at[flat_idx].max(jnp.tile(posn, S))
    winner = aux[flat_idx].reshape(S, STEP) == posn[None, :]
    ind_eff = jnp.where(winner, ind, DUMP).reshape(S, NW, 1, CHUNK)

    pars = pars_ind_steps.astype(i32).transpose(0, 2, 1).reshape(S, 2 * STEP)
    pars = pars.reshape(S, NW, 2, CHUNK)

    maskp = jnp.concatenate(
        [mask_idx.astype(i32), jnp.full((M_PAD - M,), DUMP, i32)])

    padc = jnp.zeros((NP - N, 1), f32)
    posp = jnp.concatenate([pos[:, None], padc])
    negp = jnp.concatenate([neg[:, None], padc])
    tgtp = jnp.concatenate([target[:, None], padc])

    b1r = b1_all[:, None, :]
    b2r = b2_all[:, None, :]
    r_steps = rule_steps.astype(i32)
    vref = jax.new_ref(vecpad)

    cnt = _sc_count(maskp)

    parents = _sc_gather_step(vref, pars[0])
    for t in range(S):
        nr = _mlp(r_steps[t][None], parents, parents,
                  W1_all, W1_all, b1r, W2_all, b2r)
        _sc_scatter_step(vref, nr, ind_eff[t])
        if t + 1 < S:
            parents = _sc_gather_step(vref, pars[t + 1])

    vec_final = jax.freeze(vref)
    out3 = _loss(vec_final, posp, negp, tgtp, cnt[:, None],
                 eval_W, eval_b)
    return (out3[0], out3[1], out3[2])


# speedup vs baseline: 2.1718x; 1.1603x over previous
import functools

import jax
import jax.numpy as jnp
from jax import lax
from jax.experimental import pallas as pl
from jax.experimental.pallas import tpu as pltpu
from jax.experimental.pallas import tpu_sc as plsc

N = 100000
EMB = 128
R = 8
S = 16
STEP = 4096
M = 50000
POS_WEIGHT = 2.0

NP = 100352
NP2 = NP
DUMP = N
NC = 2
NS = 16
NW = NC * NS
CHUNK = 128

RNG = NP // NW
M_PAD = 50176

_mesh = plsc.VectorSubcoreMesh(core_axis_name="c", subcore_axis_name="s")


def _wid():
    return lax.axis_index("s") * NC + lax.axis_index("c")


@functools.partial(
    pl.kernel, mesh=_mesh,
    out_type=jax.ShapeDtypeStruct((2 * STEP, EMB), jnp.float32),
    scratch_types=[
        pltpu.VMEM((2, CHUNK), jnp.int32),
        pltpu.VMEM((2 * CHUNK, EMB), jnp.float32),
        pltpu.SemaphoreType.DMA,
        pltpu.SemaphoreType.DMA,
    ],
)
def _sc_gather_step(vec_hbm, idx_hbm, out_hbm, idx_v, rows_v, s0, s1):
    w = _wid()
    pltpu.sync_copy(idx_hbm.at[w], idx_v)
    c0 = pltpu.async_copy(vec_hbm.at[idx_v.at[0]], rows_v.at[pl.ds(0, CHUNK)], s0)
    c1 = pltpu.async_copy(vec_hbm.at[idx_v.at[1]], rows_v.at[pl.ds(CHUNK, CHUNK)], s1)
    c0.wait()
    c1.wait()
    pltpu.sync_copy(rows_v, out_hbm.at[pl.ds(w * 2 * CHUNK, 2 * CHUNK)])


@functools.partial(
    pl.kernel, mesh=_mesh,
    out_type=(),
    scratch_types=[
        pltpu.VMEM((1, CHUNK), jnp.int32),
        pltpu.VMEM((CHUNK, EMB), jnp.float32),
        pltpu.SemaphoreType.DMA,
    ],
)
def _sc_scatter_step(vec_hbm, rows_hbm, idx_hbm, idx_v, rows_v, s0):
    w = _wid()
    pltpu.sync_copy(idx_hbm.at[w], idx_v)
    pltpu.sync_copy(rows_hbm.at[pl.ds(w * CHUNK, CHUNK)], rows_v)
    pltpu.async_copy(rows_v, vec_hbm.at[idx_v.at[0]], s0).wait()


@functools.partial(
    pl.kernel, mesh=_mesh,
    out_type=jax.ShapeDtypeStruct((S, STEP), jnp.int32),
    scratch_types=[
        pltpu.VMEM((NP2,), jnp.int32),
        pltpu.VMEM((STEP,), jnp.int32),
        pltpu.VMEM((STEP,), jnp.int32),
    ],
    compiler_params=pltpu.CompilerParams(needs_layout_passes=False),
)
def _sc_dedup(ind_hbm, neg1_hbm, out_hbm, stamp_v, idx_v, out_v):
    w = _wid()

    @pl.when(w < S)
    def _():
        pltpu.sync_copy(ind_hbm.at[w], idx_v)
        pltpu.sync_copy(neg1_hbm, stamp_v)
        lanes = lax.iota(jnp.int32, 16)

        def p1(k, _):
            idx = idx_v[pl.ds(k * 16, 16)]
            pos = k * 16 + lanes

            def cond(active):
                return jnp.max(active.astype(jnp.int32)) > 0

            def body(active):
                plsc.store_scatter(stamp_v, [idx], pos, mask=active)
                got = plsc.load_gather(stamp_v, [idx])
                return active & (pos > got)

            lax.while_loop(cond, body, jnp.ones((16,), jnp.bool_))
            return 0

        lax.fori_loop(0, STEP // 16, p1, 0)

        def p2(k, _):
            idx = idx_v[pl.ds(k * 16, 16)]
            pos = k * 16 + lanes
            got = plsc.load_gather(stamp_v, [idx])
            out_v[pl.ds(k * 16, 16)] = jnp.where(got == pos, idx, DUMP)
            return 0

        lax.fori_loop(0, STEP // 16, p2, 0)
        pltpu.sync_copy(out_v, out_hbm.at[w])


@functools.partial(
    pl.kernel, mesh=_mesh,
    out_type=jax.ShapeDtypeStruct((NP2,), jnp.float32),
    scratch_types=[
        pltpu.VMEM((M_PAD,), jnp.int32),
        pltpu.VMEM((RNG,), jnp.float32),
    ],
    compiler_params=pltpu.CompilerParams(needs_layout_passes=False),
)
def _sc_count(mask_hbm, cnt_hbm, mask_v, cnt_v):
    w = _wid()
    lo = w * RNG
    pltpu.sync_copy(mask_hbm, mask_v)

    def zbody(k, _):
        cnt_v[pl.ds(k * 16, 16)] = jnp.zeros((16,), jnp.float32)
        return 0

    lax.fori_loop(0, RNG // 16, zbody, 0)

    def body(k, _):
        base = k * 128
        for u in range(8):
            idx = mask_v[pl.ds(base + u * 16, 16)]
            inr = (idx >= lo) & (idx < lo + RNG)
            lidx = jnp.where(inr, idx - lo, 0)
            plsc.addupdate_scatter(
                cnt_v, [lidx], jnp.where(inr, 1.0, 0.0))
        return 0

    lax.fori_loop(0, M_PAD // 128, body, 0)
    pltpu.sync_copy(cnt_v, cnt_hbm.at[pl.ds(lo, RNG)])


_MLP_BLK = 1024


def _mlp_body(r_ref, t_ref, b_ref, w1t_ref, w1b_ref, b1_ref, w2_ref, b2_ref,
              o_ref):
    del r_ref
    h = jnp.tanh(
        jnp.dot(t_ref[...], w1t_ref[0], preferred_element_type=jnp.float32)
        + jnp.dot(b_ref[...], w1b_ref[0], preferred_element_type=jnp.float32)
        + b1_ref[0, 0]
    )
    o_ref[...] = (
        jnp.dot(h, w2_ref[0], preferred_element_type=jnp.float32)
        + b2_ref[0, 0]
    )


_mlp = pl.pallas_call(
    _mlp_body,
    grid_spec=pltpu.PrefetchScalarGridSpec(
        num_scalar_prefetch=1,
        grid=(STEP // _MLP_BLK,),
        in_specs=[
            pl.BlockSpec((_MLP_BLK, EMB), lambda i, r: (i, 0)),
            pl.BlockSpec((_MLP_BLK, EMB), lambda i, r: (i + STEP // _MLP_BLK, 0)),
            pl.BlockSpec((1, EMB, EMB), lambda i, r: (r[0], 0, 0)),
            pl.BlockSpec((1, EMB, EMB), lambda i, r: (r[0], 1, 0)),
            pl.BlockSpec((1, 1, EMB), lambda i, r: (r[0], 0, 0)),
            pl.BlockSpec((1, EMB, EMB), lambda i, r: (r[0], 0, 0)),
            pl.BlockSpec((1, 1, EMB), lambda i, r: (r[0], 0, 0)),
        ],
        out_specs=pl.BlockSpec((_MLP_BLK, EMB), lambda i, r: (i, 0)),
    ),
    out_shape=jax.ShapeDtypeStruct((STEP, EMB), jnp.float32),
)


_LOSS_BLK = 2048


def _loss_body(vec_ref, p_ref, n_ref, t_ref, c_ref, ew_ref, eb_ref, o_ref):
    i = pl.program_id(0)
    vals = (
        jnp.dot(vec_ref[...], ew_ref[...], preferred_element_type=jnp.float32)
        + eb_ref[0]
    )
    cp = c_ref[...] * p_ref[...]
    cn = c_ref[...] * n_ref[...]
    t = t_ref[...]
    sg = jnp.clip(jax.nn.sigmoid(vals), 1e-07, 1.0 - 1e-07)
    contrib = -POS_WEIGHT * t * jnp.log(sg) - (1.0 - t) * jnp.log(1.0 - sg)
    part_loss = jnp.sum((cp + cn) * contrib)
    part_pos = jnp.sum(cp * (vals >= 0.0).astype(jnp.float32))
    part_neg = jnp.sum(cn * (vals < 0.0).astype(jnp.float32))

    @pl.when(i == 0)
    def _():
        o_ref[0] = 0.0
        o_ref[1] = 0.0
        o_ref[2] = 0.0

    o_ref[0] += part_loss
    o_ref[1] += part_pos
    o_ref[2] += part_neg


_loss = pl.pallas_call(
    _loss_body,
    grid=(NP2 // _LOSS_BLK,),
    in_specs=[
        pl.BlockSpec((_LOSS_BLK, EMB), lambda i: (i, 0)),
        pl.BlockSpec((_LOSS_BLK, 1), lambda i: (i, 0)),
        pl.BlockSpec((_LOSS_BLK, 1), lambda i: (i, 0)),
        pl.BlockSpec((_LOSS_BLK, 1), lambda i: (i, 0)),
        pl.BlockSpec((_LOSS_BLK, 1), lambda i: (i, 0)),
        pl.BlockSpec((EMB, 1), lambda i: (0, 0)),
        pl.BlockSpec(memory_space=pltpu.SMEM),
    ],
    out_specs=pl.BlockSpec(memory_space=pltpu.SMEM),
    out_shape=jax.ShapeDtypeStruct((3,), jnp.float32),
)


def kernel(vectors_init, W1_all, b1_all, W2_all, b2_all, eval_W, eval_b,
           pos, neg, target, rule_steps, ind_steps, pars_ind_steps, mask_idx):
    i32 = jnp.int32
    f32 = jnp.float32

    vecpad = jnp.concatenate(
        [vectors_init, jnp.zeros((NP - N, EMB), f32)], axis=0)

    ind = ind_steps.astype(i32)
    neg1 = jnp.full((NP2,), -1, i32)

    pars = pars_ind_steps.astype(i32).transpose(0, 2, 1).reshape(S, 2 * STEP)
    pars = pars.reshape(S, NW, 2, CHUNK)

    maskp = jnp.concatenate(
        [mask_idx.astype(i32), jnp.full((M_PAD - M,), DUMP, i32)])

    padc = jnp.zeros((NP - N, 1), f32)
    posp = jnp.concatenate([pos[:, None], padc])
    negp = jnp.concatenate([neg[:, None], padc])
    tgtp = jnp.concatenate([target[:, None], padc])

    b1r = b1_all[:, None, :]
    b2r = b2_all[:, None, :]
    r_steps = rule_steps.astype(i32)
    vref = jax.new_ref(vecpad)

    parents = _sc_gather_step(vref, pars[0])
    ind_eff = _sc_dedup(ind, neg1).reshape(S, NW, 1, CHUNK)
    cnt = _sc_count(maskp)

    for t in range(S):
        nr = _mlp(r_steps[t][None], parents, parents,
                  W1_all, W1_all, b1r, W2_all, b2r)
        _sc_scatter_step(vref, nr, ind_eff[t])
        if t + 1 < S:
            parents = _sc_gather_step(vref, pars[t + 1])

    vec_final = jax.freeze(vref)
    out3 = _loss(vec_final, posp, negp, tgtp, cnt[:, None],
                 eval_W, eval_b)
    return (out3[0], out3[1], out3[2])


# speedup vs baseline: 2.9237x; 1.3462x over previous
import functools

import jax
import jax.numpy as jnp
from jax import lax
from jax.experimental import pallas as pl
from jax.experimental.pallas import tpu as pltpu
from jax.experimental.pallas import tpu_sc as plsc

N = 100000
EMB = 128
R = 8
S = 16
STEP = 4096
M = 50000
POS_WEIGHT = 2.0

NP = 100352
NP2 = NP
DUMP = N
NC = 2
NS = 16
NW = NC * NS
CHUNK = 128

RNG = NP // NW
M_PAD = 50176

_mesh = plsc.VectorSubcoreMesh(core_axis_name="c", subcore_axis_name="s")


def _wid():
    return lax.axis_index("s") * NC + lax.axis_index("c")


@functools.partial(
    pl.kernel, mesh=_mesh,
    out_type=jax.ShapeDtypeStruct((2 * STEP, EMB), jnp.float32),
    scratch_types=[
        pltpu.VMEM((2, CHUNK), jnp.int32),
        pltpu.VMEM((2 * CHUNK, EMB), jnp.float32),
        pltpu.SemaphoreType.DMA,
        pltpu.SemaphoreType.DMA,
    ],
)
def _sc_gather_step(vec_hbm, idx_hbm, out_hbm, idx_v, rows_v, s0, s1):
    w = _wid()
    pltpu.sync_copy(idx_hbm.at[w], idx_v)
    c0 = pltpu.async_copy(vec_hbm.at[idx_v.at[0]], rows_v.at[pl.ds(0, CHUNK)], s0)
    c1 = pltpu.async_copy(vec_hbm.at[idx_v.at[1]], rows_v.at[pl.ds(CHUNK, CHUNK)], s1)
    c0.wait()
    c1.wait()
    pltpu.sync_copy(rows_v, out_hbm.at[pl.ds(w * 2 * CHUNK, 2 * CHUNK)])


@functools.partial(
    pl.kernel, mesh=_mesh,
    out_type=(),
    scratch_types=[
        pltpu.VMEM((1, CHUNK), jnp.int32),
        pltpu.VMEM((CHUNK, EMB), jnp.float32),
        pltpu.SemaphoreType.DMA,
    ],
)
def _sc_scatter_step(vec_hbm, rows_hbm, idx_hbm, idx_v, rows_v, s0):
    w = _wid()
    pltpu.sync_copy(idx_hbm.at[w], idx_v)
    pltpu.sync_copy(rows_hbm.at[pl.ds(w * CHUNK, CHUNK)], rows_v)
    pltpu.async_copy(rows_v, vec_hbm.at[idx_v.at[0]], s0).wait()


@functools.partial(
    pl.kernel, mesh=_mesh,
    out_type=jax.ShapeDtypeStruct((S, STEP), jnp.int32),
    scratch_types=[
        pltpu.VMEM((NP2,), jnp.int32),
        pltpu.VMEM((STEP,), jnp.int32),
        pltpu.VMEM((STEP,), jnp.int32),
    ],
    compiler_params=pltpu.CompilerParams(needs_layout_passes=False),
)
def _sc_dedup(ind_hbm, neg1_hbm, out_hbm, stamp_v, idx_v, out_v):
    w = _wid()

    @pl.when(w < S)
    def _():
        pltpu.sync_copy(ind_hbm.at[w], idx_v)
        pltpu.sync_copy(neg1_hbm, stamp_v)
        lanes = lax.iota(jnp.int32, 16)

        def p1(k, _):
            idx = idx_v[pl.ds(k * 16, 16)]
            pos = k * 16 + lanes

            def cond(active):
                return jnp.max(active.astype(jnp.int32)) > 0

            def body(active):
                plsc.store_scatter(stamp_v, [idx], pos, mask=active)
                got = plsc.load_gather(stamp_v, [idx])
                return active & (pos > got)

            lax.while_loop(cond, body, jnp.ones((16,), jnp.bool_))
            return 0

        lax.fori_loop(0, STEP // 16, p1, 0)

        def p2(k, _):
            idx = idx_v[pl.ds(k * 16, 16)]
            pos = k * 16 + lanes
            got = plsc.load_gather(stamp_v, [idx])
            out_v[pl.ds(k * 16, 16)] = jnp.where(got == pos, idx, DUMP)
            return 0

        lax.fori_loop(0, STEP // 16, p2, 0)
        pltpu.sync_copy(out_v, out_hbm.at[w])


@functools.partial(
    pl.kernel, mesh=_mesh,
    out_type=jax.ShapeDtypeStruct((NP2,), jnp.float32),
    scratch_types=[
        pltpu.VMEM((M_PAD,), jnp.int32),
        pltpu.VMEM((RNG,), jnp.float32),
    ],
    compiler_params=pltpu.CompilerParams(needs_layout_passes=False),
)
def _sc_count(mask_hbm, cnt_hbm, mask_v, cnt_v):
    w = _wid()
    lo = w * RNG
    pltpu.sync_copy(mask_hbm, mask_v)

    def zbody(k, _):
        cnt_v[pl.ds(k * 16, 16)] = jnp.zeros((16,), jnp.float32)
        return 0

    lax.fori_loop(0, RNG // 16, zbody, 0)

    def body(k, _):
        base = k * 128
        for u in range(8):
            idx = mask_v[pl.ds(base + u * 16, 16)]
            inr = (idx >= lo) & (idx < lo + RNG)
            lidx = jnp.where(inr, idx - lo, 0)
            plsc.addupdate_scatter(
                cnt_v, [lidx], jnp.where(inr, 1.0, 0.0))
        return 0

    lax.fori_loop(0, M_PAD // 128, body, 0)
    pltpu.sync_copy(cnt_v, cnt_hbm.at[pl.ds(lo, RNG)])


_MLP_BLK = 1024


def _mlp_body(r_ref, t_ref, b_ref, w1t_ref, w1b_ref, b1_ref, w2_ref, b2_ref,
              o_ref):
    del r_ref
    h = jnp.tanh(
        jnp.dot(t_ref[...], w1t_ref[0], preferred_element_type=jnp.float32)
        + jnp.dot(b_ref[...], w1b_ref[0], preferred_element_type=jnp.float32)
        + b1_ref[0, 0]
    )
    o_ref[...] = (
        jnp.dot(h, w2_ref[0], preferred_element_type=jnp.float32)
        + b2_ref[0, 0]
    )


_mlp = pl.pallas_call(
    _mlp_body,
    grid_spec=pltpu.PrefetchScalarGridSpec(
        num_scalar_prefetch=1,
        grid=(STEP // _MLP_BLK,),
        in_specs=[
            pl.BlockSpec((_MLP_BLK, EMB), lambda i, r: (i, 0)),
            pl.BlockSpec((_MLP_BLK, EMB), lambda i, r: (i + STEP // _MLP_BLK, 0)),
            pl.BlockSpec((1, EMB, EMB), lambda i, r: (r[0], 0, 0)),
            pl.BlockSpec((1, EMB, EMB), lambda i, r: (r[0], 1, 0)),
            pl.BlockSpec((1, 1, EMB), lambda i, r: (r[0], 0, 0)),
            pl.BlockSpec((1, EMB, EMB), lambda i, r: (r[0], 0, 0)),
            pl.BlockSpec((1, 1, EMB), lambda i, r: (r[0], 0, 0)),
        ],
        out_specs=pl.BlockSpec((_MLP_BLK, EMB), lambda i, r: (i, 0)),
    ),
    out_shape=jax.ShapeDtypeStruct((STEP, EMB), jnp.float32),
)


_LOSS_BLK = 2048


def _loss_body(vec_ref, p_ref, n_ref, t_ref, c_ref, ew_ref, eb_ref, o_ref):
    i = pl.program_id(0)
    vals = (
        jnp.dot(vec_ref[...], ew_ref[...], preferred_element_type=jnp.float32)
        + eb_ref[0]
    )
    v = vals.reshape(_LOSS_BLK // EMB, EMB)
    cp = c_ref[...] * p_ref[...]
    cn = c_ref[...] * n_ref[...]
    t = t_ref[...]
    sg = jnp.clip(jax.nn.sigmoid(v), 1e-07, 1.0 - 1e-07)
    contrib = -POS_WEIGHT * t * jnp.log(sg) - (1.0 - t) * jnp.log(1.0 - sg)
    part_loss = jnp.sum((cp + cn) * contrib)
    part_pos = jnp.sum(cp * (v >= 0.0).astype(jnp.float32))
    part_neg = jnp.sum(cn * (v < 0.0).astype(jnp.float32))

    @pl.when(i == 0)
    def _():
        o_ref[0] = 0.0
        o_ref[1] = 0.0
        o_ref[2] = 0.0

    o_ref[0] += part_loss
    o_ref[1] += part_pos
    o_ref[2] += part_neg


_AUX_BLK = _LOSS_BLK // EMB

_loss = pl.pallas_call(
    _loss_body,
    grid=(NP2 // _LOSS_BLK,),
    in_specs=[
        pl.BlockSpec((_LOSS_BLK, EMB), lambda i: (i, 0)),
        pl.BlockSpec((_AUX_BLK, EMB), lambda i: (i, 0)),
        pl.BlockSpec((_AUX_BLK, EMB), lambda i: (i, 0)),
        pl.BlockSpec((_AUX_BLK, EMB), lambda i: (i, 0)),
        pl.BlockSpec((_AUX_BLK, EMB), lambda i: (i, 0)),
        pl.BlockSpec((EMB, 1), lambda i: (0, 0)),
        pl.BlockSpec(memory_space=pltpu.SMEM),
    ],
    out_specs=pl.BlockSpec(memory_space=pltpu.SMEM),
    out_shape=jax.ShapeDtypeStruct((3,), jnp.float32),
)


def kernel(vectors_init, W1_all, b1_all, W2_all, b2_all, eval_W, eval_b,
           pos, neg, target, rule_steps, ind_steps, pars_ind_steps, mask_idx):
    i32 = jnp.int32
    f32 = jnp.float32

    vecpad = jnp.concatenate(
        [vectors_init, jnp.zeros((NP - N, EMB), f32)], axis=0)

    ind = ind_steps.astype(i32)
    neg1 = jnp.full((NP2,), -1, i32)

    pars = pars_ind_steps.astype(i32).transpose(0, 2, 1).reshape(S, 2 * STEP)
    pars = pars.reshape(S, NW, 2, CHUNK)

    maskp = jnp.concatenate(
        [mask_idx.astype(i32), jnp.full((M_PAD - M,), DUMP, i32)])

    padc = jnp.zeros((NP - N,), f32)
    posp = jnp.concatenate([pos, padc]).reshape(NP // EMB, EMB)
    negp = jnp.concatenate([neg, padc]).reshape(NP // EMB, EMB)
    tgtp = jnp.concatenate([target, padc]).reshape(NP // EMB, EMB)

    b1r = b1_all[:, None, :]
    b2r = b2_all[:, None, :]
    r_steps = rule_steps.astype(i32)
    vref = jax.new_ref(vecpad)

    parents = _sc_gather_step(vref, pars[0])
    ind_eff = _sc_dedup(ind, neg1).reshape(S, NW, 1, CHUNK)
    cnt = _sc_count(maskp)

    for t in range(S):
        nr = _mlp(r_steps[t][None], parents, parents,
                  W1_all, W1_all, b1r, W2_all, b2r)
        _sc_scatter_step(vref, nr, ind_eff[t])
        if t + 1 < S:
            parents = _sc_gather_step(vref, pars[t + 1])

    vec_final = jax.freeze(vref)
    out3 = _loss(vec_final, posp, negp, tgtp,
                 cnt.reshape(NP // EMB, EMB), eval_W, eval_b)
    return (out3[0], out3[1], out3[2])


# speedup vs baseline: 3.4751x; 1.1886x over previous
import functools

import jax
import jax.numpy as jnp
from jax import lax
from jax.experimental import pallas as pl
from jax.experimental.pallas import tpu as pltpu
from jax.experimental.pallas import tpu_sc as plsc

N = 100000
EMB = 128
R = 8
S = 16
STEP = 4096
M = 50000
POS_WEIGHT = 2.0

NP = 102400
NP2 = NP
DUMP = N
NC = 2
NS = 16
NW = NC * NS
CHUNK = 128

CQ = 4
CQR = NP // CQ
CG = 8
CGP = 50176 // CG
M_PAD = 50176

_mesh = plsc.VectorSubcoreMesh(core_axis_name="c", subcore_axis_name="s")


def _wid():
    return lax.axis_index("s") * NC + lax.axis_index("c")


@functools.partial(
    pl.kernel, mesh=_mesh,
    out_type=jax.ShapeDtypeStruct((2 * STEP, EMB), jnp.float32),
    scratch_types=[
        pltpu.VMEM((2, CHUNK), jnp.int32),
        pltpu.VMEM((2 * CHUNK, EMB), jnp.float32),
        pltpu.SemaphoreType.DMA,
        pltpu.SemaphoreType.DMA,
    ],
)
def _sc_gather_step(vec_hbm, idx_hbm, out_hbm, idx_v, rows_v, s0, s1):
    w = _wid()
    pltpu.sync_copy(idx_hbm.at[w], idx_v)
    c0 = pltpu.async_copy(vec_hbm.at[idx_v.at[0]], rows_v.at[pl.ds(0, CHUNK)], s0)
    c1 = pltpu.async_copy(vec_hbm.at[idx_v.at[1]], rows_v.at[pl.ds(CHUNK, CHUNK)], s1)
    c0.wait()
    o0 = pltpu.async_copy(
        rows_v.at[pl.ds(0, CHUNK)], out_hbm.at[pl.ds(w * 2 * CHUNK, CHUNK)], s0)
    c1.wait()
    o1 = pltpu.async_copy(
        rows_v.at[pl.ds(CHUNK, CHUNK)],
        out_hbm.at[pl.ds(w * 2 * CHUNK + CHUNK, CHUNK)], s1)
    o0.wait()
    o1.wait()


@functools.partial(
    pl.kernel, mesh=_mesh,
    out_type=(),
    scratch_types=[
        pltpu.VMEM((1, CHUNK), jnp.int32),
        pltpu.VMEM((CHUNK, EMB), jnp.float32),
        pltpu.SemaphoreType.DMA,
        pltpu.SemaphoreType.DMA,
    ],
)
def _sc_scatter_step(vec_hbm, rows_hbm, idx_hbm, idx_v, rows_v, s0, s1):
    w = _wid()
    a = pltpu.async_copy(idx_hbm.at[w], idx_v, s0)
    b = pltpu.async_copy(rows_hbm.at[pl.ds(w * CHUNK, CHUNK)], rows_v, s1)
    a.wait()
    b.wait()
    pltpu.async_copy(rows_v, vec_hbm.at[idx_v.at[0]], s0).wait()


@functools.partial(
    pl.kernel, mesh=_mesh,
    out_type=jax.ShapeDtypeStruct((S, STEP), jnp.int32),
    scratch_types=[
        pltpu.VMEM((NP2,), jnp.int32),
        pltpu.VMEM((STEP,), jnp.int32),
        pltpu.VMEM((STEP,), jnp.int32),
    ],
    compiler_params=pltpu.CompilerParams(needs_layout_passes=False),
)
def _sc_dedup(ind_hbm, neg1_hbm, out_hbm, stamp_v, idx_v, out_v):
    w = _wid()

    @pl.when(w < S)
    def _():
        pltpu.sync_copy(ind_hbm.at[w], idx_v)
        pltpu.sync_copy(neg1_hbm, stamp_v)
        lanes = lax.iota(jnp.int32, 16)

        def p1(k, _):
            idx = idx_v[pl.ds(k * 16, 16)]
            pos = k * 16 + lanes

            def cond(active):
                return jnp.max(active.astype(jnp.int32)) > 0

            def body(active):
                plsc.store_scatter(stamp_v, [idx], pos, mask=active)
                got = plsc.load_gather(stamp_v, [idx])
                return active & (pos > got)

            lax.while_loop(cond, body, jnp.ones((16,), jnp.bool_))
            return 0

        lax.fori_loop(0, STEP // 16, p1, 0)

        def p2(k, _):
            idx = idx_v[pl.ds(k * 16, 16)]
            pos = k * 16 + lanes
            got = plsc.load_gather(stamp_v, [idx])
            out_v[pl.ds(k * 16, 16)] = jnp.where(got == pos, idx, DUMP)
            return 0

        lax.fori_loop(0, STEP // 16, p2, 0)
        pltpu.sync_copy(out_v, out_hbm.at[w])


@functools.partial(
    pl.kernel, mesh=_mesh,
    out_type=jax.ShapeDtypeStruct((CG, NP2 // EMB, EMB), jnp.float32),
    scratch_types=[
        pltpu.VMEM((CGP,), jnp.int32),
        pltpu.VMEM((CQR // EMB, EMB), jnp.float32),
    ],
    compiler_params=pltpu.CompilerParams(needs_layout_passes=False),
)
def _sc_count(mask_hbm, zero_hbm, cnt_hbm, mask_v, cnt_v):
    w = _wid()
    g = w // CQ
    q = w % CQ
    lo = q * CQR
    pltpu.sync_copy(mask_hbm.at[pl.ds(g * CGP, CGP)], mask_v)
    pltpu.sync_copy(zero_hbm, cnt_v)

    def body(k, _):
        base = k * 128
        for u in range(8):
            idx = mask_v[pl.ds(base + u * 16, 16)]
            inr = (idx >= lo) & (idx < lo + CQR)
            lidx = jnp.where(inr, idx - lo, 0)
            plsc.addupdate_scatter(
                cnt_v, [lidx >> 7, lidx & 127], jnp.where(inr, 1.0, 0.0))
        return 0

    lax.fori_loop(0, CGP // 128, body, 0)
    pltpu.sync_copy(
        cnt_v, cnt_hbm.at[g].at[pl.ds(q * (CQR // EMB), CQR // EMB)])


_MLP_BLK = 2048


def _mlp_body(r_ref, t_ref, b_ref, w1t_ref, w1b_ref, b1_ref, w2_ref, b2_ref,
              o_ref):
    del r_ref
    h = jnp.tanh(
        jnp.dot(t_ref[...], w1t_ref[0], preferred_element_type=jnp.float32)
        + jnp.dot(b_ref[...], w1b_ref[0], preferred_element_type=jnp.float32)
        + b1_ref[0, 0]
    )
    o_ref[...] = (
        jnp.dot(h, w2_ref[0], preferred_element_type=jnp.float32)
        + b2_ref[0, 0]
    )


_mlp = pl.pallas_call(
    _mlp_body,
    grid_spec=pltpu.PrefetchScalarGridSpec(
        num_scalar_prefetch=1,
        grid=(STEP // _MLP_BLK,),
        in_specs=[
            pl.BlockSpec((_MLP_BLK, EMB), lambda i, r: (i, 0)),
            pl.BlockSpec((_MLP_BLK, EMB), lambda i, r: (i + STEP // _MLP_BLK, 0)),
            pl.BlockSpec((1, EMB, EMB), lambda i, r: (r[0], 0, 0)),
            pl.BlockSpec((1, EMB, EMB), lambda i, r: (r[0], 1, 0)),
            pl.BlockSpec((1, 1, EMB), lambda i, r: (r[0], 0, 0)),
            pl.BlockSpec((1, EMB, EMB), lambda i, r: (r[0], 0, 0)),
            pl.BlockSpec((1, 1, EMB), lambda i, r: (r[0], 0, 0)),
        ],
        out_specs=pl.BlockSpec((_MLP_BLK, EMB), lambda i, r: (i, 0)),
    ),
    out_shape=jax.ShapeDtypeStruct((STEP, EMB), jnp.float32),
)


_LOSS_BLK = 2048


def _loss_body(vec_ref, p_ref, n_ref, t_ref, c_ref, ew_ref, eb_ref, o_ref):
    i = pl.program_id(0)
    vals = (
        jnp.dot(vec_ref[...], ew_ref[...], preferred_element_type=jnp.float32)
        + eb_ref[0]
    )
    v = vals.reshape(_LOSS_BLK // EMB, EMB)
    c = jnp.sum(c_ref[...], axis=0)
    cp = c * p_ref[...]
    cn = c * n_ref[...]
    t = t_ref[...]
    sg = jnp.clip(jax.nn.sigmoid(v), 1e-07, 1.0 - 1e-07)
    contrib = -POS_WEIGHT * t * jnp.log(sg) - (1.0 - t) * jnp.log(1.0 - sg)
    part_loss = jnp.sum((cp + cn) * contrib)
    part_pos = jnp.sum(cp * (v >= 0.0).astype(jnp.float32))
    part_neg = jnp.sum(cn * (v < 0.0).astype(jnp.float32))

    @pl.when(i == 0)
    def _():
        o_ref[0] = 0.0
        o_ref[1] = 0.0
        o_ref[2] = 0.0

    o_ref[0] += part_loss
    o_ref[1] += part_pos
    o_ref[2] += part_neg


_AUX_BLK = _LOSS_BLK // EMB

_loss = pl.pallas_call(
    _loss_body,
    grid=(NP2 // _LOSS_BLK,),
    in_specs=[
        pl.BlockSpec((_LOSS_BLK, EMB), lambda i: (i, 0)),
        pl.BlockSpec((_AUX_BLK, EMB), lambda i: (i, 0)),
        pl.BlockSpec((_AUX_BLK, EMB), lambda i: (i, 0)),
        pl.BlockSpec((_AUX_BLK, EMB), lambda i: (i, 0)),
        pl.BlockSpec((CG, _AUX_BLK, EMB), lambda i: (0, i, 0)),
        pl.BlockSpec((EMB, 1), lambda i: (0, 0)),
        pl.BlockSpec(memory_space=pltpu.SMEM),
    ],
    out_specs=pl.BlockSpec(memory_space=pltpu.SMEM),
    out_shape=jax.ShapeDtypeStruct((3,), jnp.float32),
)


def kernel(vectors_init, W1_all, b1_all, W2_all, b2_all, eval_W, eval_b,
           pos, neg, target, rule_steps, ind_steps, pars_ind_steps, mask_idx):
    i32 = jnp.int32
    f32 = jnp.float32

    vecpad = jnp.concatenate(
        [vectors_init, jnp.zeros((NP - N, EMB), f32)], axis=0)

    ind = ind_steps.astype(i32)
    neg1 = jnp.full((NP2,), -1, i32)

    pars = pars_ind_steps.astype(i32).transpose(0, 2, 1).reshape(S, 2 * STEP)
    pars = pars.reshape(S, NW, 2, CHUNK)

    maskp = jnp.concatenate(
        [mask_idx.astype(i32), jnp.full((M_PAD - M,), DUMP, i32)])

    padc = jnp.zeros((NP - N,), f32)
    posp = jnp.concatenate([pos, padc]).reshape(NP // EMB, EMB)
    negp = jnp.concatenate([neg, padc]).reshape(NP // EMB, EMB)
    tgtp = jnp.concatenate([target, padc]).reshape(NP // EMB, EMB)

    b1r = b1_all[:, None, :]
    b2r = b2_all[:, None, :]
    r_steps = rule_steps.astype(i32)
    vref = jax.new_ref(vecpad)

    ind_eff = _sc_dedup(ind, neg1).reshape(S, NW, 1, CHUNK)
    cnt = _sc_count(maskp, jnp.zeros((CQR // EMB, EMB), f32))
    parents = _sc_gather_step(vref, pars[0])

    for t in range(S):
        nr = _mlp(r_steps[t][None], parents, parents,
                  W1_all, W1_all, b1r, W2_all, b2r)
        _sc_scatter_step(vref, nr, ind_eff[t])
        if t + 1 < S:
            parents = _sc_gather_step(vref, pars[t + 1])

    vec_final = jax.freeze(vref)
    out3 = _loss(vec_final, posp, negp, tgtp, cnt, eval_W, eval_b)
    return (out3[0], out3[1], out3[2])


# speedup vs baseline: 3.7513x; 1.0795x over previous
import functools

import jax
import jax.numpy as jnp
from jax import lax
from jax.experimental import pallas as pl
from jax.experimental.pallas import tpu as pltpu
from jax.experimental.pallas import tpu_sc as plsc

N = 100000
EMB = 128
R = 8
S = 16
STEP = 4096
M = 50000
POS_WEIGHT = 2.0

NP = 102400
NP2 = NP
DUMP = N
NC = 2
NS = 16
NW = NC * NS
CHUNK = 128

CQ = 4
CQR = NP // CQ
CG = 8
CGP = 50176 // CG
M_PAD = 50176

_mesh = plsc.VectorSubcoreMesh(core_axis_name="c", subcore_axis_name="s")


def _wid():
    return lax.axis_index("s") * NC + lax.axis_index("c")


@functools.partial(
    pl.kernel, mesh=_mesh,
    out_type=jax.ShapeDtypeStruct((2 * STEP, EMB), jnp.float32),
    scratch_types=[
        pltpu.VMEM((2, CHUNK), jnp.int32),
        pltpu.VMEM((2 * CHUNK, EMB), jnp.float32),
        pltpu.SemaphoreType.DMA,
        pltpu.SemaphoreType.DMA,
    ],
)
def _sc_gather_step(vec_hbm, idx_hbm, out_hbm, idx_v, rows_v, s0, s1):
    w = _wid()
    pltpu.sync_copy(idx_hbm.at[w], idx_v)
    c0 = pltpu.async_copy(vec_hbm.at[idx_v.at[0]], rows_v.at[pl.ds(0, CHUNK)], s0)
    c1 = pltpu.async_copy(vec_hbm.at[idx_v.at[1]], rows_v.at[pl.ds(CHUNK, CHUNK)], s1)
    c0.wait()
    o0 = pltpu.async_copy(
        rows_v.at[pl.ds(0, CHUNK)], out_hbm.at[pl.ds(w * 2 * CHUNK, CHUNK)], s0)
    c1.wait()
    o1 = pltpu.async_copy(
        rows_v.at[pl.ds(CHUNK, CHUNK)],
        out_hbm.at[pl.ds(w * 2 * CHUNK + CHUNK, CHUNK)], s1)
    o0.wait()
    o1.wait()


@functools.partial(
    pl.kernel, mesh=_mesh,
    out_type=(),
    scratch_types=[
        pltpu.VMEM((1, CHUNK), jnp.int32),
        pltpu.VMEM((CHUNK, EMB), jnp.float32),
        pltpu.SemaphoreType.DMA,
        pltpu.SemaphoreType.DMA,
    ],
)
def _sc_scatter_step(vec_hbm, rows_hbm, idx_hbm, idx_v, rows_v, s0, s1):
    w = _wid()
    a = pltpu.async_copy(idx_hbm.at[w], idx_v, s0)
    b = pltpu.async_copy(rows_hbm.at[pl.ds(w * CHUNK, CHUNK)], rows_v, s1)
    a.wait()
    b.wait()
    pltpu.async_copy(rows_v, vec_hbm.at[idx_v.at[0]], s0).wait()


@functools.partial(
    pl.kernel, mesh=_mesh,
    out_type=jax.ShapeDtypeStruct((S, STEP), jnp.int32),
    scratch_types=[
        pltpu.VMEM((NP2,), jnp.int32),
        pltpu.VMEM((STEP,), jnp.int32),
        pltpu.VMEM((STEP,), jnp.int32),
    ],
    compiler_params=pltpu.CompilerParams(needs_layout_passes=False),
)
def _sc_dedup(ind_hbm, neg1_hbm, out_hbm, stamp_v, idx_v, out_v):
    w = _wid()

    @pl.when(w < S)
    def _():
        pltpu.sync_copy(ind_hbm.at[w], idx_v)
        pltpu.sync_copy(neg1_hbm, stamp_v)
        lanes = lax.iota(jnp.int32, 16)

        def p1(k, _):
            idx = idx_v[pl.ds(k * 16, 16)]
            pos = k * 16 + lanes

            def cond(active):
                return jnp.max(active.astype(jnp.int32)) > 0

            def body(active):
                plsc.store_scatter(stamp_v, [idx], pos, mask=active)
                got = plsc.load_gather(stamp_v, [idx])
                return active & (pos > got)

            lax.while_loop(cond, body, jnp.ones((16,), jnp.bool_))
            return 0

        lax.fori_loop(0, STEP // 16, p1, 0)

        def p2(k, _):
            idx = idx_v[pl.ds(k * 16, 16)]
            pos = k * 16 + lanes
            got = plsc.load_gather(stamp_v, [idx])
            out_v[pl.ds(k * 16, 16)] = jnp.where(got == pos, idx, DUMP)
            return 0

        lax.fori_loop(0, STEP // 16, p2, 0)
        pltpu.sync_copy(out_v, out_hbm.at[w])


@functools.partial(
    pl.kernel, mesh=_mesh,
    out_type=jax.ShapeDtypeStruct((CG, NP2 // EMB, EMB), jnp.float32),
    scratch_types=[
        pltpu.VMEM((CGP,), jnp.int32),
        pltpu.VMEM((CQR // EMB, EMB), jnp.float32),
    ],
    compiler_params=pltpu.CompilerParams(needs_layout_passes=False),
)
def _sc_count(mask_hbm, zero_hbm, cnt_hbm, mask_v, cnt_v):
    w = _wid()
    g = w // CQ
    q = w % CQ
    lo = q * CQR
    pltpu.sync_copy(mask_hbm.at[pl.ds(g * CGP, CGP)], mask_v)
    pltpu.sync_copy(zero_hbm, cnt_v)

    def body(k, _):
        base = k * 128
        for u in range(8):
            idx = mask_v[pl.ds(base + u * 16, 16)]
            inr = (idx >= lo) & (idx < lo + CQR)
            lidx = jnp.where(inr, idx - lo, 0)
            plsc.addupdate_scatter(
                cnt_v, [lidx >> 7, lidx & 127], jnp.where(inr, 1.0, 0.0))
        return 0

    lax.fori_loop(0, CGP // 128, body, 0)
    pltpu.sync_copy(
        cnt_v, cnt_hbm.at[g].at[pl.ds(q * (CQR // EMB), CQR // EMB)])


_MLP_BLK = 2048


def _mlp_body(r_ref, t_ref, b_ref, w1t_ref, w1b_ref, b1_ref, w2_ref, b2_ref,
              o_ref):
    del r_ref
    h = jnp.tanh(
        jnp.dot(t_ref[...], w1t_ref[0], preferred_element_type=jnp.float32)
        + jnp.dot(b_ref[...], w1b_ref[0], preferred_element_type=jnp.float32)
        + b1_ref[0, 0]
    )
    o_ref[...] = (
        jnp.dot(h, w2_ref[0], preferred_element_type=jnp.float32)
        + b2_ref[0, 0]
    )


_mlp = pl.pallas_call(
    _mlp_body,
    grid_spec=pltpu.PrefetchScalarGridSpec(
        num_scalar_prefetch=1,
        grid=(STEP // _MLP_BLK,),
        in_specs=[
            pl.BlockSpec((_MLP_BLK, EMB), lambda i, r: (i, 0)),
            pl.BlockSpec((_MLP_BLK, EMB), lambda i, r: (i + STEP // _MLP_BLK, 0)),
            pl.BlockSpec((1, EMB, EMB), lambda i, r: (r[0], 0, 0)),
            pl.BlockSpec((1, EMB, EMB), lambda i, r: (r[0], 1, 0)),
            pl.BlockSpec((1, 1, EMB), lambda i, r: (r[0], 0, 0)),
            pl.BlockSpec((1, EMB, EMB), lambda i, r: (r[0], 0, 0)),
            pl.BlockSpec((1, 1, EMB), lambda i, r: (r[0], 0, 0)),
        ],
        out_specs=pl.BlockSpec((_MLP_BLK, EMB), lambda i, r: (i, 0)),
    ),
    out_shape=jax.ShapeDtypeStruct((STEP, EMB), jnp.float32),
)


_LOSS_BLK = 2048


def _loss_body(vec_ref, p_ref, n_ref, t_ref, c_ref, ew_ref, eb_ref, o_ref):
    i = pl.program_id(0)
    x3 = vec_ref[...].reshape(_LOSS_BLK // EMB, EMB, EMB)
    v = lax.dot_general(
        ew_ref[...][:, 0], x3, (((0,), (2,)), ((), ())),
        preferred_element_type=jnp.float32,
    ) + eb_ref[0]
    c = jnp.sum(c_ref[...], axis=0)
    cp = c * p_ref[...]
    cn = c * n_ref[...]
    t = t_ref[...]
    sg = jnp.clip(jax.nn.sigmoid(v), 1e-07, 1.0 - 1e-07)
    contrib = -POS_WEIGHT * t * jnp.log(sg) - (1.0 - t) * jnp.log(1.0 - sg)
    part_loss = jnp.sum((cp + cn) * contrib)
    part_pos = jnp.sum(cp * (v >= 0.0).astype(jnp.float32))
    part_neg = jnp.sum(cn * (v < 0.0).astype(jnp.float32))

    @pl.when(i == 0)
    def _():
        o_ref[0] = 0.0
        o_ref[1] = 0.0
        o_ref[2] = 0.0

    o_ref[0] += part_loss
    o_ref[1] += part_pos
    o_ref[2] += part_neg


_AUX_BLK = _LOSS_BLK // EMB

_loss = pl.pallas_call(
    _loss_body,
    grid=(NP2 // _LOSS_BLK,),
    in_specs=[
        pl.BlockSpec((_LOSS_BLK, EMB), lambda i: (i, 0)),
        pl.BlockSpec((_AUX_BLK, EMB), lambda i: (i, 0)),
        pl.BlockSpec((_AUX_BLK, EMB), lambda i: (i, 0)),
        pl.BlockSpec((_AUX_BLK, EMB), lambda i: (i, 0)),
        pl.BlockSpec((CG, _AUX_BLK, EMB), lambda i: (0, i, 0)),
        pl.BlockSpec((EMB, 1), lambda i: (0, 0)),
        pl.BlockSpec(memory_space=pltpu.SMEM),
    ],
    out_specs=pl.BlockSpec(memory_space=pltpu.SMEM),
    out_shape=jax.ShapeDtypeStruct((3,), jnp.float32),
)


def kernel(vectors_init, W1_all, b1_all, W2_all, b2_all, eval_W, eval_b,
           pos, neg, target, rule_steps, ind_steps, pars_ind_steps, mask_idx):
    i32 = jnp.int32
    f32 = jnp.float32

    vecpad = jnp.concatenate(
        [vectors_init, jnp.zeros((NP - N, EMB), f32)], axis=0)

    ind = ind_steps.astype(i32)
    neg1 = jnp.full((NP2,), -1, i32)

    pars = pars_ind_steps.astype(i32).transpose(0, 2, 1).reshape(S, 2 * STEP)
    pars = pars.reshape(S, NW, 2, CHUNK)

    maskp = jnp.concatenate(
        [mask_idx.astype(i32), jnp.full((M_PAD - M,), DUMP, i32)])

    padc = jnp.zeros((NP - N,), f32)
    posp = jnp.concatenate([pos, padc]).reshape(NP // EMB, EMB)
    negp = jnp.concatenate([neg, padc]).reshape(NP // EMB, EMB)
    tgtp = jnp.concatenate([target, padc]).reshape(NP // EMB, EMB)

    b1r = b1_all[:, None, :]
    b2r = b2_all[:, None, :]
    r_steps = rule_steps.astype(i32)
    vref = jax.new_ref(vecpad)

    ind_eff = _sc_dedup(ind, neg1).reshape(S, NW, 1, CHUNK)
    cnt = _sc_count(maskp, jnp.zeros((CQR // EMB, EMB), f32))
    parents = _sc_gather_step(vref, pars[0])

    for t in range(S):
        nr = _mlp(r_steps[t][None], parents, parents,
                  W1_all, W1_all, b1r, W2_all, b2r)
        _sc_scatter_step(vref, nr, ind_eff[t])
        if t + 1 < S:
            parents = _sc_gather_step(vref, pars[t + 1])

    vec_final = jax.freeze(vref)
    out3 = _loss(vec_final, posp, negp, tgtp, cnt, eval_W, eval_b)
    return (out3[0], out3[1], out3[2])


# speedup vs baseline: 3.8380x; 1.0231x over previous
import functools

import jax
import jax.numpy as jnp
from jax import lax
from jax.experimental import pallas as pl
from jax.experimental.pallas import tpu as pltpu
from jax.experimental.pallas import tpu_sc as plsc

N = 100000
EMB = 128
R = 8
S = 16
STEP = 4096
M = 50000
POS_WEIGHT = 2.0

NP = 102400
NP2 = NP
DUMP = N
NC = 2
NS = 16
NW = NC * NS
CHUNK = 128

CQ = 4
CQR = NP // CQ
CG = 8
CGP = 50176 // CG
M_PAD = 50176

_mesh = plsc.VectorSubcoreMesh(core_axis_name="c", subcore_axis_name="s")


def _wid():
    return lax.axis_index("s") * NC + lax.axis_index("c")


@functools.partial(
    pl.kernel, mesh=_mesh,
    out_type=jax.ShapeDtypeStruct((2 * STEP, EMB), jnp.float32),
    scratch_types=[
        pltpu.VMEM((2, CHUNK), jnp.int32),
        pltpu.VMEM((2 * CHUNK, EMB), jnp.float32),
        pltpu.SemaphoreType.DMA,
        pltpu.SemaphoreType.DMA,
    ],
)
def _sc_gather_step(vec_hbm, idx_hbm, out_hbm, idx_v, rows_v, s0, s1):
    w = _wid()
    pltpu.sync_copy(idx_hbm.at[w], idx_v)
    c0 = pltpu.async_copy(vec_hbm.at[idx_v.at[0]], rows_v.at[pl.ds(0, CHUNK)], s0)
    c1 = pltpu.async_copy(vec_hbm.at[idx_v.at[1]], rows_v.at[pl.ds(CHUNK, CHUNK)], s1)
    c0.wait()
    o0 = pltpu.async_copy(
        rows_v.at[pl.ds(0, CHUNK)], out_hbm.at[pl.ds(w * 2 * CHUNK, CHUNK)], s0)
    c1.wait()
    o1 = pltpu.async_copy(
        rows_v.at[pl.ds(CHUNK, CHUNK)],
        out_hbm.at[pl.ds(w * 2 * CHUNK + CHUNK, CHUNK)], s1)
    o0.wait()
    o1.wait()


@functools.partial(
    pl.kernel, mesh=_mesh,
    out_type=(),
    scratch_types=[
        pltpu.VMEM((1, CHUNK), jnp.int32),
        pltpu.VMEM((CHUNK, EMB), jnp.float32),
        pltpu.SemaphoreType.DMA,
        pltpu.SemaphoreType.DMA,
    ],
)
def _sc_scatter_step(vec_hbm, rows_hbm, idx_hbm, idx_v, rows_v, s0, s1):
    w = _wid()
    a = pltpu.async_copy(idx_hbm.at[w], idx_v, s0)
    b = pltpu.async_copy(rows_hbm.at[pl.ds(w * CHUNK, CHUNK)], rows_v, s1)
    a.wait()
    b.wait()
    pltpu.async_copy(rows_v, vec_hbm.at[idx_v.at[0]], s0).wait()


@functools.partial(
    pl.kernel, mesh=_mesh,
    out_type=jax.ShapeDtypeStruct((S, STEP), jnp.int32),
    scratch_types=[
        pltpu.VMEM((NP2,), jnp.int32),
        pltpu.VMEM((STEP,), jnp.int32),
        pltpu.VMEM((STEP,), jnp.int32),
    ],
    compiler_params=pltpu.CompilerParams(needs_layout_passes=False),
)
def _sc_dedup(ind_hbm, neg1_hbm, out_hbm, stamp_v, idx_v, out_v):
    w = _wid()

    @pl.when(w < S)
    def _():
        pltpu.sync_copy(ind_hbm.at[w], idx_v)
        pltpu.sync_copy(neg1_hbm, stamp_v)
        lanes = lax.iota(jnp.int32, 16)

        def p1(k, _):
            idx = idx_v[pl.ds(k * 16, 16)]
            pos = k * 16 + lanes

            def cond(active):
                return jnp.max(active.astype(jnp.int32)) > 0

            def body(active):
                plsc.store_scatter(stamp_v, [idx], pos, mask=active)
                got = plsc.load_gather(stamp_v, [idx])
                return active & (pos > got)

            lax.while_loop(cond, body, jnp.ones((16,), jnp.bool_))
            return 0

        lax.fori_loop(0, STEP // 16, p1, 0)

        def p2(k, _):
            idx = idx_v[pl.ds(k * 16, 16)]
            pos = k * 16 + lanes
            got = plsc.load_gather(stamp_v, [idx])
            out_v[pl.ds(k * 16, 16)] = jnp.where(got == pos, idx, DUMP)
            return 0

        lax.fori_loop(0, STEP // 16, p2, 0)
        pltpu.sync_copy(out_v, out_hbm.at[w])


@functools.partial(
    pl.kernel, mesh=_mesh,
    out_type=jax.ShapeDtypeStruct((CG, NP2 // EMB, EMB), jnp.float32),
    scratch_types=[
        pltpu.VMEM((CGP,), jnp.int32),
        pltpu.VMEM((CQR // EMB, EMB), jnp.float32),
    ],
    compiler_params=pltpu.CompilerParams(needs_layout_passes=False),
)
def _sc_count(mask_hbm, zero_hbm, cnt_hbm, mask_v, cnt_v):
    w = _wid()
    g = w // CQ
    q = w % CQ
    lo = q * CQR
    pltpu.sync_copy(mask_hbm.at[pl.ds(g * CGP, CGP)], mask_v)
    pltpu.sync_copy(zero_hbm, cnt_v)

    def body(k, _):
        base = k * 128
        for u in range(8):
            idx = mask_v[pl.ds(base + u * 16, 16)]
            inr = (idx >= lo) & (idx < lo + CQR)
            lidx = jnp.where(inr, idx - lo, 0)
            plsc.addupdate_scatter(
                cnt_v, [lidx >> 7, lidx & 127], jnp.where(inr, 1.0, 0.0))
        return 0

    lax.fori_loop(0, CGP // 128, body, 0)
    pltpu.sync_copy(
        cnt_v, cnt_hbm.at[g].at[pl.ds(q * (CQR // EMB), CQR // EMB)])


_MLP_BLK = 4096


def _mlp_body(r_ref, t_ref, b_ref, w1t_ref, w1b_ref, b1_ref, w2_ref, b2_ref,
              o_ref):
    del r_ref
    h = jnp.tanh(
        jnp.dot(t_ref[...], w1t_ref[0], preferred_element_type=jnp.float32)
        + jnp.dot(b_ref[...], w1b_ref[0], preferred_element_type=jnp.float32)
        + b1_ref[0, 0]
    )
    o_ref[...] = (
        jnp.dot(h, w2_ref[0], preferred_element_type=jnp.float32)
        + b2_ref[0, 0]
    )


_mlp = pl.pallas_call(
    _mlp_body,
    grid_spec=pltpu.PrefetchScalarGridSpec(
        num_scalar_prefetch=1,
        grid=(STEP // _MLP_BLK,),
        in_specs=[
            pl.BlockSpec((_MLP_BLK, EMB), lambda i, r: (i, 0)),
            pl.BlockSpec((_MLP_BLK, EMB), lambda i, r: (i + STEP // _MLP_BLK, 0)),
            pl.BlockSpec((1, EMB, EMB), lambda i, r: (r[0], 0, 0)),
            pl.BlockSpec((1, EMB, EMB), lambda i, r: (r[0], 1, 0)),
            pl.BlockSpec((1, 1, EMB), lambda i, r: (r[0], 0, 0)),
            pl.BlockSpec((1, EMB, EMB), lambda i, r: (r[0], 0, 0)),
            pl.BlockSpec((1, 1, EMB), lambda i, r: (r[0], 0, 0)),
        ],
        out_specs=pl.BlockSpec((_MLP_BLK, EMB), lambda i, r: (i, 0)),
    ),
    out_shape=jax.ShapeDtypeStruct((STEP, EMB), jnp.float32),
)


_LOSS_BLK = 4096


def _loss_body(vec_ref, p_ref, n_ref, t_ref, c_ref, ew_ref, eb_ref, o_ref):
    i = pl.program_id(0)
    x3 = vec_ref[...].reshape(_LOSS_BLK // EMB, EMB, EMB)
    v = lax.dot_general(
        ew_ref[...][:, 0], x3, (((0,), (2,)), ((), ())),
        preferred_element_type=jnp.float32,
    ) + eb_ref[0]
    c = jnp.sum(c_ref[...], axis=0)
    cp = c * p_ref[...]
    cn = c * n_ref[...]
    t = t_ref[...]
    sg = jnp.clip(jax.nn.sigmoid(v), 1e-07, 1.0 - 1e-07)
    contrib = -POS_WEIGHT * t * jnp.log(sg) - (1.0 - t) * jnp.log(1.0 - sg)
    part_loss = jnp.sum((cp + cn) * contrib)
    part_pos = jnp.sum(cp * (v >= 0.0).astype(jnp.float32))
    part_neg = jnp.sum(cn * (v < 0.0).astype(jnp.float32))

    @pl.when(i == 0)
    def _():
        o_ref[0] = 0.0
        o_ref[1] = 0.0
        o_ref[2] = 0.0

    o_ref[0] += part_loss
    o_ref[1] += part_pos
    o_ref[2] += part_neg


_AUX_BLK = _LOSS_BLK // EMB

_loss = pl.pallas_call(
    _loss_body,
    grid=(NP2 // _LOSS_BLK,),
    in_specs=[
        pl.BlockSpec((_LOSS_BLK, EMB), lambda i: (i, 0)),
        pl.BlockSpec((_AUX_BLK, EMB), lambda i: (i, 0)),
        pl.BlockSpec((_AUX_BLK, EMB), lambda i: (i, 0)),
        pl.BlockSpec((_AUX_BLK, EMB), lambda i: (i, 0)),
        pl.BlockSpec((CG, _AUX_BLK, EMB), lambda i: (0, i, 0)),
        pl.BlockSpec((EMB, 1), lambda i: (0, 0)),
        pl.BlockSpec(memory_space=pltpu.SMEM),
    ],
    out_specs=pl.BlockSpec(memory_space=pltpu.SMEM),
    out_shape=jax.ShapeDtypeStruct((3,), jnp.float32),
)


def kernel(vectors_init, W1_all, b1_all, W2_all, b2_all, eval_W, eval_b,
           pos, neg, target, rule_steps, ind_steps, pars_ind_steps, mask_idx):
    i32 = jnp.int32
    f32 = jnp.float32

    vecpad = jnp.concatenate(
        [vectors_init, jnp.zeros((NP - N, EMB), f32)], axis=0)

    ind = ind_steps.astype(i32)
    neg1 = jnp.full((NP2,), -1, i32)

    pars = pars_ind_steps.astype(i32).transpose(0, 2, 1).reshape(S, 2 * STEP)
    pars = pars.reshape(S, NW, 2, CHUNK)

    maskp = jnp.concatenate(
        [mask_idx.astype(i32), jnp.full((M_PAD - M,), DUMP, i32)])

    padc = jnp.zeros((NP - N,), f32)
    posp = jnp.concatenate([pos, padc]).reshape(NP // EMB, EMB)
    negp = jnp.concatenate([neg, padc]).reshape(NP // EMB, EMB)
    tgtp = jnp.concatenate([target, padc]).reshape(NP // EMB, EMB)

    b1r = b1_all[:, None, :]
    b2r = b2_all[:, None, :]
    r_steps = rule_steps.astype(i32)
    vref = jax.new_ref(vecpad)

    ind_eff = _sc_dedup(ind, neg1).reshape(S, NW, 1, CHUNK)
    cnt = _sc_count(maskp, jnp.zeros((CQR // EMB, EMB), f32))
    parents = _sc_gather_step(vref, pars[0])

    for t in range(S):
        nr = _mlp(r_steps[t][None], parents, parents,
                  W1_all, W1_all, b1r, W2_all, b2r)
        _sc_scatter_step(vref, nr, ind_eff[t])
        if t + 1 < S:
            parents = _sc_gather_step(vref, pars[t + 1])

    vec_final = jax.freeze(vref)
    out3 = _loss(vec_final, posp, negp, tgtp, cnt, eval_W, eval_b)
    return (out3[0], out3[1], out3[2])
